# baseline (speedup 1.0000x reference)
import functools

import jax
import jax.numpy as jnp
from jax import lax
from jax.experimental import pallas as pl
from jax.experimental.pallas import tpu as pltpu
from jax.experimental.pallas import tpu_sc as plsc

_SC_ROWS = 2048
_SC_WORKERS = 32
_SC_RPW = _SC_ROWS // _SC_WORKERS


def _sc_stream_body(labels_hbm, out_hbm, buf, sem):
    wid = lax.axis_index("s") * 2 + lax.axis_index("c")
    base = (labels_hbm.shape[0] - _SC_ROWS) + wid * _SC_RPW
    pltpu.async_copy(labels_hbm.at[pl.ds(base, _SC_RPW), :], buf, sem).wait()
    pltpu.sync_copy(buf.at[0], out_hbm.at[wid])


def _sc_stream(labels):
    mesh = plsc.VectorSubcoreMesh(core_axis_name="c", subcore_axis_name="s")
    num_labels = labels.shape[1]
    fn = functools.partial(
        pl.kernel,
        out_type=jax.ShapeDtypeStruct((_SC_WORKERS, num_labels), jnp.float32),
        mesh=mesh,
        scratch_types=[
            pltpu.VMEM((_SC_RPW, num_labels), jnp.float32),
            pltpu.SemaphoreType.DMA,
        ],
    )(_sc_stream_body)
    return fn(labels)


def _block_body(labels_ref, features_ref, centroids_ref, theta_row_ref,
                f_part_ref, cnt_part_ref):
    blk, num_labels = labels_ref.shape

    rad_row = jax.nn.softplus(theta_row_ref[...])

    lbl = labels_ref[...]
    col = lax.broadcasted_iota(jnp.int32, (blk, num_labels), 1)
    row_max = jnp.max(lbl, axis=1, keepdims=True)
    first = jnp.min(jnp.where(lbl == row_max, col, num_labels),
                    axis=1, keepdims=True)
    onehot = (col == first).astype(jnp.bfloat16)

    c = jnp.dot(onehot, centroids_ref[...],
                preferred_element_type=jnp.float32)
    diff = features_ref[...] - c
    d = jnp.sqrt(jnp.sum(diff * diff, axis=1, keepdims=True))

    cnt_part_ref[...] = jnp.sum(onehot.astype(jnp.float32), axis=0,
                                keepdims=True)[None]
    f_part_ref[...] = jnp.sum(jnp.abs(d - rad_row), axis=0, keepdims=True)[None]


def _reduce_body(f_part_ref, cnt_part_ref, theta_ref, loss_ref, radius_ref,
                 *, batch_total):
    radius_ref[...] = jax.nn.softplus(theta_ref[...])
    f_tot = jnp.sum(f_part_ref[...], axis=0)
    cnt_tot = jnp.sum(cnt_part_ref[...], axis=0)
    total = jnp.sum(f_tot * cnt_tot, axis=1, keepdims=True)
    loss_ref[...] = total / jnp.float32(batch_total * batch_total)


def kernel(features, centroids, labels, theta):
    batch, feat_dim = features.shape
    num_labels = centroids.shape[0]
    blk = 1024
    nb = batch // blk

    theta_row = theta.reshape(1, num_labels)
    centroids_bf16 = centroids.astype(jnp.bfloat16)

    f_part, cnt_part = pl.pallas_call(
        _block_body,
        grid=(nb,),
        in_specs=[
            pl.BlockSpec((blk, num_labels), lambda g: (g, 0)),
            pl.BlockSpec((blk, feat_dim), lambda g: (g, 0)),
            pl.BlockSpec((num_labels, feat_dim), lambda g: (0, 0)),
            pl.BlockSpec((1, num_labels), lambda g: (0, 0)),
        ],
        out_specs=(
            pl.BlockSpec((1, 1, num_labels), lambda g: (g, 0, 0)),
            pl.BlockSpec((1, 1, num_labels), lambda g: (g, 0, 0)),
        ),
        out_shape=(
            jax.ShapeDtypeStruct((nb, 1, num_labels), jnp.float32),
            jax.ShapeDtypeStruct((nb, 1, num_labels), jnp.float32),
        ),
        compiler_params=pltpu.CompilerParams(
            dimension_semantics=("parallel",),
        ),
    )(labels, features, centroids_bf16, theta_row)

    loss2d, radius = pl.pallas_call(
        functools.partial(_reduce_body, batch_total=batch),
        out_shape=(
            jax.ShapeDtypeStruct((1, 1), jnp.float32),
            jax.ShapeDtypeStruct((num_labels, 1), jnp.float32),
        ),
    )(f_part, cnt_part, theta)

    sc_out = _sc_stream(labels)
    loss, sc_keep = lax.optimization_barrier((loss2d[0, 0], sc_out[0, 0]))
    return loss + 0.0 * sc_keep, radius


# speedup vs baseline: 1.2627x; 1.2627x over previous
import functools

import jax
import jax.numpy as jnp
from jax import lax
from jax.experimental import pallas as pl
from jax.experimental.pallas import tpu as pltpu

_LANES = 128


def _block_body(labels_ref, features_ref, centroids_ref, theta_row_ref,
                f_part_ref, cnt_part_ref):
    blk, num_labels = labels_ref.shape
    nchunk = num_labels // _LANES
    rem = num_labels - nchunk * _LANES

    rad_row = jax.nn.softplus(theta_row_ref[...])
    cent = centroids_ref[...].astype(jnp.bfloat16)

    bases = [c * _LANES for c in range(nchunk)]
    if rem:
        bases.append(num_labels - _LANES)
    lane = lax.broadcasted_iota(jnp.int32, (blk, _LANES), 1)
    m = labels_ref[:, : _LANES]
    idxb = jnp.zeros((blk, _LANES), jnp.int32)
    for b in bases[1:]:
        v = labels_ref[:, b:b + _LANES]
        upd = v > m
        m = jnp.where(upd, v, m)
        idxb = jnp.where(upd, jnp.int32(b), idxb)
    row_max = jnp.max(m, axis=1, keepdims=True)
    first = jnp.min(jnp.where(m == row_max, idxb + lane, num_labels),
                    axis=1, keepdims=True)

    col = lax.broadcasted_iota(jnp.int32, (blk, num_labels), 1)
    onehot = (col == first).astype(jnp.bfloat16)

    c = jnp.dot(onehot, cent,
                preferred_element_type=jnp.float32)
    diff = features_ref[...] - c
    d = jnp.sqrt(jnp.sum(diff * diff, axis=1, keepdims=True))

    cnt_part_ref[...] = jnp.sum(onehot.astype(jnp.float32), axis=0,
                                keepdims=True)[None]
    f_part_ref[...] = jnp.sum(jnp.abs(d - rad_row), axis=0, keepdims=True)[None]


def _reduce_body(f_part_ref, cnt_part_ref, theta_ref, loss_ref, radius_ref,
                 *, batch_total):
    radius_ref[...] = jax.nn.softplus(theta_ref[...])
    f_tot = jnp.sum(f_part_ref[...], axis=0)
    cnt_tot = jnp.sum(cnt_part_ref[...], axis=0)
    total = jnp.sum(f_tot * cnt_tot, axis=1, keepdims=True)
    loss_ref[...] = total / jnp.float32(batch_total * batch_total)


def kernel(features, centroids, labels, theta):
    batch, feat_dim = features.shape
    num_labels = centroids.shape[0]
    blk = 1024
    nb = batch // blk

    theta_row = theta.reshape(1, num_labels)

    f_part, cnt_part = pl.pallas_call(
        _block_body,
        grid=(nb,),
        in_specs=[
            pl.BlockSpec((blk, num_labels), lambda g: (g, 0)),
            pl.BlockSpec((blk, feat_dim), lambda g: (g, 0)),
            pl.BlockSpec((num_labels, feat_dim), lambda g: (0, 0)),
            pl.BlockSpec((1, num_labels), lambda g: (0, 0)),
        ],
        out_specs=(
            pl.BlockSpec((1, 1, num_labels), lambda g: (g, 0, 0)),
            pl.BlockSpec((1, 1, num_labels), lambda g: (g, 0, 0)),
        ),
        out_shape=(
            jax.ShapeDtypeStruct((nb, 1, num_labels), jnp.float32),
            jax.ShapeDtypeStruct((nb, 1, num_labels), jnp.float32),
        ),
        compiler_params=pltpu.CompilerParams(
            dimension_semantics=("parallel",),
        ),
    )(labels, features, centroids, theta_row)

    loss2d, radius = pl.pallas_call(
        functools.partial(_reduce_body, batch_total=batch),
        out_shape=(
            jax.ShapeDtypeStruct((1, 1), jnp.float32),
            jax.ShapeDtypeStruct((num_labels, 1), jnp.float32),
        ),
    )(f_part, cnt_part, theta)

    return loss2d[0, 0], radius


# speedup vs baseline: 1.5307x; 1.2123x over previous
import functools

import jax
import jax.numpy as jnp
from jax import lax
from jax.experimental import pallas as pl
from jax.experimental.pallas import tpu as pltpu


def _block_body(labels_ref, features_ref, centroids_ref, theta_row_ref,
                f_part_ref, cnt_part_ref):
    blk, num_labels = labels_ref.shape

    rad_row = jax.nn.softplus(theta_row_ref[...])
    cent = centroids_ref[...].astype(jnp.bfloat16)

    lbl = labels_ref[...]
    col = lax.broadcasted_iota(jnp.int32, (blk, num_labels), 1)
    row_max = jnp.max(lbl, axis=1, keepdims=True)
    first = jnp.min(jnp.where(lbl == row_max, col, num_labels),
                    axis=1, keepdims=True)
    onehot = (col == first).astype(jnp.bfloat16)

    c = jnp.dot(onehot, cent,
                preferred_element_type=jnp.float32)
    diff = features_ref[...] - c
    d = jnp.sqrt(jnp.sum(diff * diff, axis=1, keepdims=True))

    cnt_part_ref[...] = jnp.sum(onehot.astype(jnp.float32), axis=0,
                                keepdims=True)[None]
    f_part_ref[...] = jnp.sum(jnp.abs(d - rad_row), axis=0, keepdims=True)[None]


def _reduce_body(f_part_ref, cnt_part_ref, theta_ref, loss_ref, radius_ref,
                 *, batch_total):
    radius_ref[...] = jax.nn.softplus(theta_ref[...])
    f_tot = jnp.sum(f_part_ref[...], axis=0)
    cnt_tot = jnp.sum(cnt_part_ref[...], axis=0)
    total = jnp.sum(f_tot * cnt_tot, axis=1, keepdims=True)
    loss_ref[...] = total / jnp.float32(batch_total * batch_total)


def kernel(features, centroids, labels, theta):
    batch, feat_dim = features.shape
    num_labels = centroids.shape[0]
    blk = 1024
    nb = batch // blk

    theta_row = theta.reshape(1, num_labels)

    f_part, cnt_part = pl.pallas_call(
        _block_body,
        grid=(nb,),
        in_specs=[
            pl.BlockSpec((blk, num_labels), lambda g: (g, 0)),
            pl.BlockSpec((blk, feat_dim), lambda g: (g, 0)),
            pl.BlockSpec((num_labels, feat_dim), lambda g: (0, 0)),
            pl.BlockSpec((1, num_labels), lambda g: (0, 0)),
        ],
        out_specs=(
            pl.BlockSpec((1, 1, num_labels), lambda g: (g, 0, 0)),
            pl.BlockSpec((1, 1, num_labels), lambda g: (g, 0, 0)),
        ),
        out_shape=(
            jax.ShapeDtypeStruct((nb, 1, num_labels), jnp.float32),
            jax.ShapeDtypeStruct((nb, 1, num_labels), jnp.float32),
        ),
        compiler_params=pltpu.CompilerParams(
            dimension_semantics=("parallel",),
        ),
    )(labels, features, centroids, theta_row)

    loss2d, radius = pl.pallas_call(
        functools.partial(_reduce_body, batch_total=batch),
        out_shape=(
            jax.ShapeDtypeStruct((1, 1), jnp.float32),
            jax.ShapeDtypeStruct((num_labels, 1), jnp.float32),
        ),
    )(f_part, cnt_part, theta)

    return loss2d[0, 0], radius
